# baseline (speedup 1.0000x reference)
import functools

import jax
import jax.numpy as jnp
import numpy as np
from jax import lax
from jax.experimental import pallas as pl
from jax.experimental.pallas import tpu as pltpu
from jax.experimental.pallas import tpu_sc as plsc

E = 3_200_000
F = 3
VOCAB = 8
EMB = 16

NC, NS = 2, 16
NW = NC * NS
CHUNK = 3200
N_CHUNKS = E // CHUNK
MAX_ITERS = -(-N_CHUNKS // NW)
_GCHUNKS = [(k * 128, 128) for k in range(CHUNK // 128)]
N_GROUPS = E // 128
GROUPS_PER_CHUNK = CHUNK // 128

_PACK_R = E // 128
_PACK_BLK = 1000
_W = np.zeros((3 * 128, 128), np.float32)
_W[3 * np.arange(128) + 0, np.arange(128)] = 64.0
_W[3 * np.arange(128) + 1, np.arange(128)] = 8.0
_W[3 * np.arange(128) + 2, np.arange(128)] = 1.0


def _combine_body(t0_ref, t1_ref, t2_ref, c_ref):
    t0 = t0_ref[...]
    t1 = t1_ref[...]
    t2 = t2_ref[...]
    x = t0[:, None, None, :] + t1[None, :, None, :] + t2[None, None, :, :]
    c_ref[...] = x.reshape(VOCAB ** 3, EMB)


def _build_combined(T0, T1, T2):
    return pl.pallas_call(
        _combine_body,
        out_shape=jax.ShapeDtypeStruct((VOCAB ** 3, EMB), jnp.float32),
    )(T0, T1, T2)


def _pack_body(attr_ref, w_ref, idx_ref):
    x = attr_ref[...].astype(jnp.float32)
    y = jax.lax.dot(x, w_ref[...], preferred_element_type=jnp.float32)
    idx_ref[...] = y.astype(jnp.int32)


def _pack_indices(edge_attr):
    flat = edge_attr.reshape(_PACK_R, 3 * 128)
    idx = pl.pallas_call(
        _pack_body,
        grid=(_PACK_R // _PACK_BLK,),
        in_specs=[
            pl.BlockSpec((_PACK_BLK, 3 * 128), lambda i: (i, 0)),
            pl.BlockSpec((3 * 128, 128), lambda i: (0, 0)),
        ],
        out_specs=pl.BlockSpec((_PACK_BLK, 128), lambda i: (i, 0)),
        out_shape=jax.ShapeDtypeStruct((_PACK_R, 128), jnp.int32),
    )(flat, jnp.asarray(_W))
    return idx.reshape(E)


@functools.partial(
    pl.kernel,
    out_type=jax.ShapeDtypeStruct((N_GROUPS, 128, EMB), jnp.float32),
    mesh=plsc.VectorSubcoreMesh(core_axis_name="c", subcore_axis_name="s"),
    compiler_params=pltpu.CompilerParams(use_tc_tiling_on_sc=False),
    scratch_types=[
        pltpu.VMEM((CHUNK,), jnp.int32),
        pltpu.VMEM((GROUPS_PER_CHUNK, 128, EMB), jnp.float32),
        pltpu.SemaphoreType.DMA,
    ],
)
def _sc_encode(idx_hbm, c_hbm, out_hbm, idx_v, rows_v, gsem):
    wid = lax.axis_index("s") * NC + lax.axis_index("c")

    def outer(i, carry):
        cid = wid + i * NW

        @pl.when(cid < N_CHUNKS)
        def _():
            pltpu.sync_copy(idx_hbm.at[pl.ds(cid * CHUNK, CHUNK)], idx_v)
            handles = [
                pltpu.async_copy(
                    c_hbm.at[idx_v.at[pl.ds(k * 128, 128)]],
                    rows_v.at[k],
                    gsem,
                )
                for k in range(GROUPS_PER_CHUNK)
            ]
            for h in handles:
                h.wait()
            pltpu.sync_copy(
                rows_v,
                out_hbm.at[pl.ds(cid * GROUPS_PER_CHUNK, GROUPS_PER_CHUNK)],
            )

        return carry

    lax.fori_loop(0, MAX_ITERS, outer, 0)


def kernel(edge_attr, T0, T1, T2):
    c = _build_combined(T0, T1, T2)
    idx = _pack_indices(edge_attr)
    return _sc_encode(idx, c).reshape(E, EMB)
